# baseline (speedup 1.0000x reference)
import functools

import jax
import jax.numpy as jnp
from jax import lax
from jax.experimental import pallas as pl
from jax.experimental.pallas import tpu as pltpu
from jax.experimental.pallas import tpu_sc as plsc

_N_BLK = 1000
_E_BLK = 4000
_CHUNK_SS = 40
_CHUNK_PG = 80
_UNROLL = 5
_DEG_W = 16


def _mm2_body(x_ref, wl_ref, wr_ref, o1_ref, o2_ref):
    x = x_ref[...]
    o1_ref[...] = jnp.dot(x, wl_ref[...].T, preferred_element_type=jnp.float32)
    o2_ref[...] = jnp.dot(x, wr_ref[...].T, preferred_element_type=jnp.float32)


def _combine1_body(sp_ref, cnt_ref, xr_ref, b_ref, w_ref, wr_ref,
                   t2_ref, hr_ref, ci_ref):
    s = sp_ref[0] + sp_ref[1]
    cnt = cnt_ref[0, :, 0:1] + cnt_ref[1, :, 0:1]
    ci = 1.0 / jnp.maximum(cnt, 1.0)
    h = jnp.maximum(s * ci + b_ref[...] + xr_ref[...], 0.0)
    t2_ref[...] = jnp.dot(h, w_ref[...].T, preferred_element_type=jnp.float32)
    hr_ref[...] = jnp.dot(h, wr_ref[...].T, preferred_element_type=jnp.float32)
    ci_ref[...] = ci


def _combine2_body(sp_ref, hr_ref, ci_ref, b_ref, wa_ref, wb_ref,
                   p_ref, q_ref):
    s = sp_ref[0] + sp_ref[1]
    emb = s * ci_ref[...] + b_ref[...] + hr_ref[...]
    p_ref[...] = jnp.dot(emb, wa_ref[...].T, preferred_element_type=jnp.float32)
    q_ref[...] = jnp.dot(emb, wb_ref[...].T, preferred_element_type=jnp.float32)


def _cls_body(pq_ref, ea_ref, wce_ref, c0_ref, w2_ref, b2_ref,
              w3_ref, b3_ref, o_ref):
    z = (pq_ref[...]
         + jnp.dot(ea_ref[...], wce_ref[...].T,
                   preferred_element_type=jnp.float32)
         + c0_ref[...])
    z = jnp.maximum(z, 0.0)
    z2 = jnp.maximum(
        jnp.dot(z, w2_ref[...].T, preferred_element_type=jnp.float32)
        + b2_ref[...], 0.0)
    o_ref[...] = (jnp.sum(z2 * w3_ref[...], axis=1, keepdims=True)
                  + b3_ref[0, 0])


def _mm2(x, wl, wr, n_blk):
    n, d = x.shape
    h = wl.shape[0]
    return pl.pallas_call(
        _mm2_body,
        grid=(n // n_blk,),
        in_specs=[pl.BlockSpec((n_blk, d), lambda i: (i, 0)),
                  pl.BlockSpec((h, d), lambda i: (0, 0)),
                  pl.BlockSpec((h, d), lambda i: (0, 0))],
        out_specs=[pl.BlockSpec((n_blk, h), lambda i: (i, 0)),
                   pl.BlockSpec((n_blk, h), lambda i: (i, 0))],
        out_shape=[jax.ShapeDtypeStruct((n, h), jnp.float32),
                   jax.ShapeDtypeStruct((n, h), jnp.float32)],
    )(x, wl, wr)


def _sc_segsum(table, src1d, dst1d, zeros, nc, ns, chunk, rpt, unroll):
    nodes_pad, d = zeros.shape
    rows_per_sub = nodes_pad // ns
    per_tile = rpt * chunk
    n_it = rpt // unroll
    mesh = plsc.VectorSubcoreMesh(core_axis_name="c", subcore_axis_name="s")

    blk = unroll * chunk
    scratch = [pltpu.VMEM((blk,), jnp.int32),
               pltpu.VMEM((blk,), jnp.int32)]
    scratch += [pltpu.VMEM((chunk, d), jnp.float32)] * unroll
    scratch += [pltpu.VMEM_SHARED((nodes_pad, d), jnp.float32)]
    scratch += [pltpu.SemaphoreType.DMA] * (2 * unroll)

    @functools.partial(
        pl.kernel, mesh=mesh,
        out_type=jax.ShapeDtypeStruct((nc, nodes_pad, d), jnp.float32),
        scratch_types=scratch)
    def k(table_hbm, src_hbm, dst_hbm, zeros_hbm, out_hbm, *scr):
        src_blk, dst_blk = scr[0], scr[1]
        rows = scr[2:2 + unroll]
        acc_sh = scr[2 + unroll]
        sems = scr[3 + unroll:3 + 2 * unroll]
        sems2 = scr[3 + 2 * unroll:]

        c = lax.axis_index("c")
        s = lax.axis_index("s")
        w = s * nc + c
        tbase = w * per_tile
        pltpu.sync_copy(zeros_hbm.at[pl.ds(s * rows_per_sub, rows_per_sub)],
                        acc_sh.at[pl.ds(s * rows_per_sub, rows_per_sub)])
        plsc.subcore_barrier()

        def body(j, carry):
            base = tbase + j * blk
            pltpu.sync_copy(src_hbm.at[pl.ds(base, blk)], src_blk)
            pltpu.sync_copy(dst_hbm.at[pl.ds(base, blk)], dst_blk)
            cps = []
            for u in range(unroll):
                cps.append(pltpu.async_copy(
                    table_hbm.at[src_blk.at[pl.ds(u * chunk, chunk)]],
                    rows[u], sems[u]))
            adds = []
            for u in range(unroll):
                cps[u].wait()
                adds.append(pltpu.async_copy(
                    rows[u],
                    acc_sh.at[dst_blk.at[pl.ds(u * chunk, chunk)]],
                    sems2[u], add=True))
            for a in adds:
                a.wait()
            return carry

        lax.fori_loop(0, n_it, body, 0)
        plsc.subcore_barrier()
        pltpu.sync_copy(acc_sh.at[pl.ds(s * rows_per_sub, rows_per_sub)],
                        out_hbm.at[c, pl.ds(s * rows_per_sub, rows_per_sub)])

    return k(table, src1d, dst1d, zeros)


def _sc_degree(dst1d, ones, zeros, nc, ns, chunk, rpt):
    nodes_pad, d = zeros.shape
    rows_per_sub = nodes_pad // ns
    per_tile = rpt * chunk
    mesh = plsc.VectorSubcoreMesh(core_axis_name="c", subcore_axis_name="s")

    @functools.partial(
        pl.kernel,
        mesh=mesh,
        out_type=jax.ShapeDtypeStruct((nc, nodes_pad, d), jnp.float32),
        scratch_types=[
            pltpu.VMEM((per_tile,), jnp.int32),
            pltpu.VMEM((chunk, d), jnp.float32),
            pltpu.VMEM_SHARED((nodes_pad, d), jnp.float32),
        ],
    )
    def k(dst_hbm, ones_hbm, zeros_hbm, out_hbm, dst_all, ones_v, acc_sh):
        c = lax.axis_index("c")
        s = lax.axis_index("s")
        w = s * nc + c
        tbase = w * per_tile
        pltpu.sync_copy(dst_hbm.at[pl.ds(tbase, per_tile)], dst_all)
        pltpu.sync_copy(ones_hbm, ones_v)
        pltpu.sync_copy(zeros_hbm.at[pl.ds(s * rows_per_sub, rows_per_sub)],
                        acc_sh.at[pl.ds(s * rows_per_sub, rows_per_sub)])
        plsc.subcore_barrier()

        def body(j, carry):
            pltpu.sync_copy(ones_v, acc_sh.at[dst_all.at[pl.ds(j * chunk, chunk)]],
                            add=True)
            return carry

        lax.fori_loop(0, rpt, body, 0)
        plsc.subcore_barrier()
        pltpu.sync_copy(acc_sh.at[pl.ds(s * rows_per_sub, rows_per_sub)],
                        out_hbm.at[c, pl.ds(s * rows_per_sub, rows_per_sub)])

    return k(dst1d, ones, zeros)


def _sc_pair_gather(p, q, src1d, dst1d, nc, ns, chunk, rpt, unroll):
    nodes, d = p.shape
    e = src1d.shape[0]
    per_tile = rpt * chunk
    n_it = rpt // unroll
    mesh = plsc.VectorSubcoreMesh(core_axis_name="c", subcore_axis_name="s")

    scratch = [pltpu.VMEM((per_tile,), jnp.int32),
               pltpu.VMEM((per_tile,), jnp.int32)]
    scratch += [pltpu.VMEM((chunk, d), jnp.float32)] * unroll
    scratch += [pltpu.SemaphoreType.DMA] * (2 * unroll)

    @functools.partial(
        pl.kernel, mesh=mesh,
        out_type=jax.ShapeDtypeStruct((e, d), jnp.float32),
        scratch_types=scratch)
    def k(p_hbm, q_hbm, src_hbm, dst_hbm, pq_hbm, *scr):
        src_all, dst_all = scr[0], scr[1]
        pq_v = scr[2:2 + unroll]
        sp = scr[2 + unroll:2 + 2 * unroll]
        sq = scr[2 + 2 * unroll:]

        c = lax.axis_index("c")
        s = lax.axis_index("s")
        w = s * nc + c
        tbase = w * per_tile
        pltpu.sync_copy(src_hbm.at[pl.ds(tbase, per_tile)], src_all)
        pltpu.sync_copy(dst_hbm.at[pl.ds(tbase, per_tile)], dst_all)

        def body(j, carry):
            cps = []
            for u in range(unroll):
                off = (j * unroll + u) * chunk
                cps.append(pltpu.async_copy(
                    p_hbm.at[src_all.at[pl.ds(off, chunk)]], pq_v[u], sp[u]))
            adds = []
            for u in range(unroll):
                off = (j * unroll + u) * chunk
                cps[u].wait()
                adds.append(pltpu.async_copy(
                    q_hbm.at[dst_all.at[pl.ds(off, chunk)]], pq_v[u], sq[u],
                    add=True))
            wrs = []
            for u in range(unroll):
                off = (j * unroll + u) * chunk
                adds[u].wait()
                wrs.append(pltpu.async_copy(
                    pq_v[u], pq_hbm.at[pl.ds(tbase + off, chunk)], sp[u]))
            for h in wrs:
                h.wait()
            return carry

        lax.fori_loop(0, n_it, body, 0)

    return k(p, q, src1d, dst1d)


def kernel(x, edge_index, edge_attr, W1l, b1l, W1r, W2l, b2l, W2r,
           Wc1, bc1, gamma, beta, Wc2, bc2, Wc3, bc3):
    n, d = x.shape
    e = edge_index.shape[1]
    h_dim = W1l.shape[0]
    de = edge_attr.shape[1]

    info = plsc.get_sparse_core_info()
    nc, ns = info.num_cores, info.num_subcores
    nw = nc * ns
    chunk_ss = _CHUNK_SS
    chunk_pg = _CHUNK_PG
    rpt_ss = e // (chunk_ss * nw)
    rpt_pg = e // (chunk_pg * nw)
    n_pad = ((n + ns * 8 - 1) // (ns * 8)) * (ns * 8)

    src1d = edge_index[0]
    dst1d = edge_index[1]

    s_bn = 1.0 / jnp.sqrt(jnp.float32(1.0 + 1e-5))
    g = gamma * s_bn
    wa = Wc1[:, :h_dim] * g[:, None]
    wb = Wc1[:, h_dim:2 * h_dim] * g[:, None]
    wce = Wc1[:, 2 * h_dim:] * g[:, None]
    c0 = (bc1 * g + beta).reshape(1, h_dim)
    b1l_r = b1l.reshape(1, h_dim)
    b2l_r = b2l.reshape(1, h_dim)
    bc2_r = bc2.reshape(1, h_dim // 2)
    bc3_r = bc3.reshape(1, 1)

    zeros_h = jnp.zeros((n_pad, h_dim), jnp.float32)
    zeros_d = jnp.zeros((n_pad, _DEG_W), jnp.float32)
    ones_chunk = jnp.ones((chunk_ss, h_dim), jnp.float32)

    t1, xr = _mm2(x, W1l, W1r, _N_BLK)
    cntp = _sc_degree(dst1d, ones_chunk, zeros_h, nc, ns, chunk_ss, rpt_ss)
    s1p = _sc_segsum(t1, src1d, dst1d, zeros_h, nc, ns, chunk_ss, rpt_ss,
                     _UNROLL)

    t2, hr, ci = pl.pallas_call(
        _combine1_body,
        grid=(n // _N_BLK,),
        in_specs=[pl.BlockSpec((nc, _N_BLK, h_dim), lambda i: (0, i, 0)),
                  pl.BlockSpec((nc, _N_BLK, h_dim), lambda i: (0, i, 0)),
                  pl.BlockSpec((_N_BLK, h_dim), lambda i: (i, 0)),
                  pl.BlockSpec((1, h_dim), lambda i: (0, 0)),
                  pl.BlockSpec((h_dim, h_dim), lambda i: (0, 0)),
                  pl.BlockSpec((h_dim, h_dim), lambda i: (0, 0))],
        out_specs=[pl.BlockSpec((_N_BLK, h_dim), lambda i: (i, 0)),
                   pl.BlockSpec((_N_BLK, h_dim), lambda i: (i, 0)),
                   pl.BlockSpec((_N_BLK, 1), lambda i: (i, 0))],
        out_shape=[jax.ShapeDtypeStruct((n, h_dim), jnp.float32),
                   jax.ShapeDtypeStruct((n, h_dim), jnp.float32),
                   jax.ShapeDtypeStruct((n, 1), jnp.float32)],
    )(s1p, cntp, xr, b1l_r, W2l, W2r)

    s2p = _sc_segsum(t2, src1d, dst1d, zeros_h, nc, ns, chunk_ss, rpt_ss,
                     _UNROLL)

    p_tab, q_tab = pl.pallas_call(
        _combine2_body,
        grid=(n // _N_BLK,),
        in_specs=[pl.BlockSpec((nc, _N_BLK, h_dim), lambda i: (0, i, 0)),
                  pl.BlockSpec((_N_BLK, h_dim), lambda i: (i, 0)),
                  pl.BlockSpec((_N_BLK, 1), lambda i: (i, 0)),
                  pl.BlockSpec((1, h_dim), lambda i: (0, 0)),
                  pl.BlockSpec((h_dim, h_dim), lambda i: (0, 0)),
                  pl.BlockSpec((h_dim, h_dim), lambda i: (0, 0))],
        out_specs=[pl.BlockSpec((_N_BLK, h_dim), lambda i: (i, 0)),
                   pl.BlockSpec((_N_BLK, h_dim), lambda i: (i, 0))],
        out_shape=[jax.ShapeDtypeStruct((n, h_dim), jnp.float32),
                   jax.ShapeDtypeStruct((n, h_dim), jnp.float32)],
    )(s2p, hr, ci, b2l_r, wa, wb)

    pq = _sc_pair_gather(p_tab, q_tab, src1d, dst1d,
                         nc, ns, chunk_pg, rpt_pg, _UNROLL)

    logits2d = pl.pallas_call(
        _cls_body,
        grid=(e // _E_BLK,),
        in_specs=[pl.BlockSpec((_E_BLK, h_dim), lambda i: (i, 0)),
                  pl.BlockSpec((_E_BLK, de), lambda i: (i, 0)),
                  pl.BlockSpec((h_dim, de), lambda i: (0, 0)),
                  pl.BlockSpec((1, h_dim), lambda i: (0, 0)),
                  pl.BlockSpec((h_dim // 2, h_dim), lambda i: (0, 0)),
                  pl.BlockSpec((1, h_dim // 2), lambda i: (0, 0)),
                  pl.BlockSpec((1, h_dim // 2), lambda i: (0, 0)),
                  pl.BlockSpec((1, 1), lambda i: (0, 0))],
        out_specs=pl.BlockSpec((_E_BLK, 1), lambda i: (i, 0)),
        out_shape=jax.ShapeDtypeStruct((e, 1), jnp.float32),
    )(pq, edge_attr, wce, c0, Wc2, bc2_r, Wc3, bc3_r)

    return logits2d[:, 0]


# speedup vs baseline: 1.0103x; 1.0103x over previous
import functools

import jax
import jax.numpy as jnp
from jax import lax
from jax.experimental import pallas as pl
from jax.experimental.pallas import tpu as pltpu
from jax.experimental.pallas import tpu_sc as plsc

_N_BLK = 1000
_E_BLK = 4000
_CHUNK_SS = 40
_CHUNK_PG = 80
_UNROLL = 5
_DEG_W = 16


def _mm_body(x_ref, w_ref, o_ref):
    o_ref[...] = jnp.dot(x_ref[...], w_ref[...].T,
                         preferred_element_type=jnp.float32)


def _combine1_body(sp_ref, cnt_ref, xr_ref, b_ref, w_ref, h_ref, t2_ref,
                   ci_ref):
    s = sp_ref[0] + sp_ref[1]
    cnt = cnt_ref[0, :, 0:1] + cnt_ref[1, :, 0:1]
    ci = 1.0 / jnp.maximum(cnt, 1.0)
    h = jnp.maximum(s * ci + b_ref[...] + xr_ref[...], 0.0)
    h_ref[...] = h
    t2_ref[...] = jnp.dot(h, w_ref[...].T, preferred_element_type=jnp.float32)
    ci_ref[...] = ci


def _combine2_body(sp_ref, hr_ref, ci_ref, b_ref, wa_ref, wb_ref,
                   p_ref, q_ref):
    s = sp_ref[0] + sp_ref[1]
    emb = s * ci_ref[...] + b_ref[...] + hr_ref[...]
    p_ref[...] = jnp.dot(emb, wa_ref[...].T, preferred_element_type=jnp.float32)
    q_ref[...] = jnp.dot(emb, wb_ref[...].T, preferred_element_type=jnp.float32)


def _cls_body(pq_ref, ea_ref, wce_ref, c0_ref, w2_ref, b2_ref,
              w3_ref, b3_ref, o_ref):
    z = (pq_ref[...]
         + jnp.dot(ea_ref[...], wce_ref[...].T,
                   preferred_element_type=jnp.float32)
         + c0_ref[...])
    z = jnp.maximum(z, 0.0)
    z2 = jnp.maximum(
        jnp.dot(z, w2_ref[...].T, preferred_element_type=jnp.float32)
        + b2_ref[...], 0.0)
    o_ref[...] = (jnp.sum(z2 * w3_ref[...], axis=1, keepdims=True)
                  + b3_ref[0, 0])


def _mm(x, w, n_blk):
    n, d = x.shape
    h = w.shape[0]
    return pl.pallas_call(
        _mm_body,
        grid=(n // n_blk,),
        in_specs=[pl.BlockSpec((n_blk, d), lambda i: (i, 0)),
                  pl.BlockSpec((h, d), lambda i: (0, 0))],
        out_specs=pl.BlockSpec((n_blk, h), lambda i: (i, 0)),
        out_shape=jax.ShapeDtypeStruct((n, h), jnp.float32),
    )(x, w)


def _sc_segsum(table, src1d, dst1d, zeros, nc, ns, chunk, rpt, unroll):
    nodes_pad, d = zeros.shape
    rows_per_sub = nodes_pad // ns
    per_tile = rpt * chunk
    n_it = rpt // unroll
    mesh = plsc.VectorSubcoreMesh(core_axis_name="c", subcore_axis_name="s")

    blk = unroll * chunk
    scratch = [pltpu.VMEM((blk,), jnp.int32),
               pltpu.VMEM((blk,), jnp.int32)]
    scratch += [pltpu.VMEM((chunk, d), jnp.float32)] * unroll
    scratch += [pltpu.VMEM_SHARED((nodes_pad, d), jnp.float32)]
    scratch += [pltpu.SemaphoreType.DMA] * (2 * unroll)

    @functools.partial(
        pl.kernel, mesh=mesh,
        out_type=jax.ShapeDtypeStruct((nc, nodes_pad, d), jnp.float32),
        scratch_types=scratch)
    def k(table_hbm, src_hbm, dst_hbm, zeros_hbm, out_hbm, *scr):
        src_blk, dst_blk = scr[0], scr[1]
        rows = scr[2:2 + unroll]
        acc_sh = scr[2 + unroll]
        sems = scr[3 + unroll:3 + 2 * unroll]
        sems2 = scr[3 + 2 * unroll:]

        c = lax.axis_index("c")
        s = lax.axis_index("s")
        w = s * nc + c
        tbase = w * per_tile
        pltpu.sync_copy(zeros_hbm.at[pl.ds(s * rows_per_sub, rows_per_sub)],
                        acc_sh.at[pl.ds(s * rows_per_sub, rows_per_sub)])
        plsc.subcore_barrier()

        def body(j, carry):
            base = tbase + j * blk
            pltpu.sync_copy(src_hbm.at[pl.ds(base, blk)], src_blk)
            pltpu.sync_copy(dst_hbm.at[pl.ds(base, blk)], dst_blk)
            cps = []
            for u in range(unroll):
                cps.append(pltpu.async_copy(
                    table_hbm.at[src_blk.at[pl.ds(u * chunk, chunk)]],
                    rows[u], sems[u]))
            adds = []
            for u in range(unroll):
                cps[u].wait()
                adds.append(pltpu.async_copy(
                    rows[u],
                    acc_sh.at[dst_blk.at[pl.ds(u * chunk, chunk)]],
                    sems2[u], add=True))
            for a in adds:
                a.wait()
            return carry

        lax.fori_loop(0, n_it, body, 0)
        plsc.subcore_barrier()
        pltpu.sync_copy(acc_sh.at[pl.ds(s * rows_per_sub, rows_per_sub)],
                        out_hbm.at[c, pl.ds(s * rows_per_sub, rows_per_sub)])

    return k(table, src1d, dst1d, zeros)


def _sc_degree(dst1d, ones, zeros, nc, ns, chunk, rpt):
    nodes_pad, d = zeros.shape
    rows_per_sub = nodes_pad // ns
    per_tile = rpt * chunk
    mesh = plsc.VectorSubcoreMesh(core_axis_name="c", subcore_axis_name="s")

    @functools.partial(
        pl.kernel,
        mesh=mesh,
        out_type=jax.ShapeDtypeStruct((nc, nodes_pad, d), jnp.float32),
        scratch_types=[
            pltpu.VMEM((per_tile,), jnp.int32),
            pltpu.VMEM((chunk, d), jnp.float32),
            pltpu.VMEM_SHARED((nodes_pad, d), jnp.float32),
        ],
    )
    def k(dst_hbm, ones_hbm, zeros_hbm, out_hbm, dst_all, ones_v, acc_sh):
        c = lax.axis_index("c")
        s = lax.axis_index("s")
        w = s * nc + c
        tbase = w * per_tile
        pltpu.sync_copy(dst_hbm.at[pl.ds(tbase, per_tile)], dst_all)
        pltpu.sync_copy(ones_hbm, ones_v)
        pltpu.sync_copy(zeros_hbm.at[pl.ds(s * rows_per_sub, rows_per_sub)],
                        acc_sh.at[pl.ds(s * rows_per_sub, rows_per_sub)])
        plsc.subcore_barrier()

        def body(j, carry):
            pltpu.sync_copy(ones_v, acc_sh.at[dst_all.at[pl.ds(j * chunk, chunk)]],
                            add=True)
            return carry

        lax.fori_loop(0, rpt, body, 0)
        plsc.subcore_barrier()
        pltpu.sync_copy(acc_sh.at[pl.ds(s * rows_per_sub, rows_per_sub)],
                        out_hbm.at[c, pl.ds(s * rows_per_sub, rows_per_sub)])

    return k(dst1d, ones, zeros)


def _sc_pair_gather(p, q, src1d, dst1d, nc, ns, chunk, rpt, unroll):
    nodes, d = p.shape
    e = src1d.shape[0]
    per_tile = rpt * chunk
    n_it = rpt // unroll
    mesh = plsc.VectorSubcoreMesh(core_axis_name="c", subcore_axis_name="s")

    scratch = [pltpu.VMEM((per_tile,), jnp.int32),
               pltpu.VMEM((per_tile,), jnp.int32)]
    scratch += [pltpu.VMEM((chunk, d), jnp.float32)] * unroll
    scratch += [pltpu.SemaphoreType.DMA] * (2 * unroll)

    @functools.partial(
        pl.kernel, mesh=mesh,
        out_type=jax.ShapeDtypeStruct((e, d), jnp.float32),
        scratch_types=scratch)
    def k(p_hbm, q_hbm, src_hbm, dst_hbm, pq_hbm, *scr):
        src_all, dst_all = scr[0], scr[1]
        pq_v = scr[2:2 + unroll]
        sp = scr[2 + unroll:2 + 2 * unroll]
        sq = scr[2 + 2 * unroll:]

        c = lax.axis_index("c")
        s = lax.axis_index("s")
        w = s * nc + c
        tbase = w * per_tile
        pltpu.sync_copy(src_hbm.at[pl.ds(tbase, per_tile)], src_all)
        pltpu.sync_copy(dst_hbm.at[pl.ds(tbase, per_tile)], dst_all)

        def body(j, carry):
            cps = []
            for u in range(unroll):
                off = (j * unroll + u) * chunk
                cps.append(pltpu.async_copy(
                    p_hbm.at[src_all.at[pl.ds(off, chunk)]], pq_v[u], sp[u]))
            adds = []
            for u in range(unroll):
                off = (j * unroll + u) * chunk
                cps[u].wait()
                adds.append(pltpu.async_copy(
                    q_hbm.at[dst_all.at[pl.ds(off, chunk)]], pq_v[u], sq[u],
                    add=True))
            wrs = []
            for u in range(unroll):
                off = (j * unroll + u) * chunk
                adds[u].wait()
                wrs.append(pltpu.async_copy(
                    pq_v[u], pq_hbm.at[pl.ds(tbase + off, chunk)], sp[u]))
            for h in wrs:
                h.wait()
            return carry

        lax.fori_loop(0, n_it, body, 0)

    return k(p, q, src1d, dst1d)


def kernel(x, edge_index, edge_attr, W1l, b1l, W1r, W2l, b2l, W2r,
           Wc1, bc1, gamma, beta, Wc2, bc2, Wc3, bc3):
    n, d = x.shape
    e = edge_index.shape[1]
    h_dim = W1l.shape[0]
    de = edge_attr.shape[1]

    info = plsc.get_sparse_core_info()
    nc, ns = info.num_cores, info.num_subcores
    nw = nc * ns
    chunk_ss = _CHUNK_SS
    chunk_pg = _CHUNK_PG
    rpt_ss = e // (chunk_ss * nw)
    rpt_pg = e // (chunk_pg * nw)
    n_pad = ((n + ns * 8 - 1) // (ns * 8)) * (ns * 8)

    src1d = edge_index[0]
    dst1d = edge_index[1]

    s_bn = 1.0 / jnp.sqrt(jnp.float32(1.0 + 1e-5))
    g = gamma * s_bn
    wa = Wc1[:, :h_dim] * g[:, None]
    wb = Wc1[:, h_dim:2 * h_dim] * g[:, None]
    wce = Wc1[:, 2 * h_dim:] * g[:, None]
    c0 = (bc1 * g + beta).reshape(1, h_dim)
    b1l_r = b1l.reshape(1, h_dim)
    b2l_r = b2l.reshape(1, h_dim)
    bc2_r = bc2.reshape(1, h_dim // 2)
    bc3_r = bc3.reshape(1, 1)

    zeros_h = jnp.zeros((n_pad, h_dim), jnp.float32)
    zeros_d = jnp.zeros((n_pad, _DEG_W), jnp.float32)
    ones_chunk = jnp.ones((chunk_ss, h_dim), jnp.float32)

    t1 = _mm(x, W1l, _N_BLK)
    cntp = _sc_degree(dst1d, ones_chunk, zeros_h, nc, ns, chunk_ss, rpt_ss)
    s1p = _sc_segsum(t1, src1d, dst1d, zeros_h, nc, ns, chunk_ss, rpt_ss,
                     _UNROLL)
    xr = _mm(x, W1r, _N_BLK)

    h, t2, ci = pl.pallas_call(
        _combine1_body,
        grid=(n // _N_BLK,),
        in_specs=[pl.BlockSpec((nc, _N_BLK, h_dim), lambda i: (0, i, 0)),
                  pl.BlockSpec((nc, _N_BLK, h_dim), lambda i: (0, i, 0)),
                  pl.BlockSpec((_N_BLK, h_dim), lambda i: (i, 0)),
                  pl.BlockSpec((1, h_dim), lambda i: (0, 0)),
                  pl.BlockSpec((h_dim, h_dim), lambda i: (0, 0))],
        out_specs=[pl.BlockSpec((_N_BLK, h_dim), lambda i: (i, 0)),
                   pl.BlockSpec((_N_BLK, h_dim), lambda i: (i, 0)),
                   pl.BlockSpec((_N_BLK, 1), lambda i: (i, 0))],
        out_shape=[jax.ShapeDtypeStruct((n, h_dim), jnp.float32),
                   jax.ShapeDtypeStruct((n, h_dim), jnp.float32),
                   jax.ShapeDtypeStruct((n, 1), jnp.float32)],
    )(s1p, cntp, xr, b1l_r, W2l)

    s2p = _sc_segsum(t2, src1d, dst1d, zeros_h, nc, ns, chunk_ss, rpt_ss,
                     _UNROLL)
    hr = _mm(h, W2r, _N_BLK)

    p_tab, q_tab = pl.pallas_call(
        _combine2_body,
        grid=(n // _N_BLK,),
        in_specs=[pl.BlockSpec((nc, _N_BLK, h_dim), lambda i: (0, i, 0)),
                  pl.BlockSpec((_N_BLK, h_dim), lambda i: (i, 0)),
                  pl.BlockSpec((_N_BLK, 1), lambda i: (i, 0)),
                  pl.BlockSpec((1, h_dim), lambda i: (0, 0)),
                  pl.BlockSpec((h_dim, h_dim), lambda i: (0, 0)),
                  pl.BlockSpec((h_dim, h_dim), lambda i: (0, 0))],
        out_specs=[pl.BlockSpec((_N_BLK, h_dim), lambda i: (i, 0)),
                   pl.BlockSpec((_N_BLK, h_dim), lambda i: (i, 0))],
        out_shape=[jax.ShapeDtypeStruct((n, h_dim), jnp.float32),
                   jax.ShapeDtypeStruct((n, h_dim), jnp.float32)],
    )(s2p, hr, ci, b2l_r, wa, wb)

    pq = _sc_pair_gather(p_tab, q_tab, src1d, dst1d,
                         nc, ns, chunk_pg, rpt_pg, _UNROLL)

    logits2d = pl.pallas_call(
        _cls_body,
        grid=(e // _E_BLK,),
        in_specs=[pl.BlockSpec((_E_BLK, h_dim), lambda i: (i, 0)),
                  pl.BlockSpec((_E_BLK, de), lambda i: (i, 0)),
                  pl.BlockSpec((h_dim, de), lambda i: (0, 0)),
                  pl.BlockSpec((1, h_dim), lambda i: (0, 0)),
                  pl.BlockSpec((h_dim // 2, h_dim), lambda i: (0, 0)),
                  pl.BlockSpec((1, h_dim // 2), lambda i: (0, 0)),
                  pl.BlockSpec((1, h_dim // 2), lambda i: (0, 0)),
                  pl.BlockSpec((1, 1), lambda i: (0, 0))],
        out_specs=pl.BlockSpec((_E_BLK, 1), lambda i: (i, 0)),
        out_shape=jax.ShapeDtypeStruct((e, 1), jnp.float32),
    )(pq, edge_attr, wce, c0, Wc2, bc2_r, Wc3, bc3_r)

    return logits2d[:, 0]
